# baseline (speedup 1.0000x reference)
import functools

import jax
import jax.numpy as jnp
from jax import lax
from jax.experimental import pallas as pl
from jax.experimental.pallas import tpu as pltpu
from jax.experimental.pallas import tpu_sc as plsc

_ROWS = 160000
_COLS = 256
_NC = 2
_NS = 16
_NW = _NC * _NS
_ROWS_PER = _ROWS // _NW
_CHUNK = 200
_NCHUNK = _ROWS_PER // _CHUNK


def kernel(x_j):
    mesh = plsc.VectorSubcoreMesh(core_axis_name="c", subcore_axis_name="s")

    @functools.partial(
        pl.kernel,
        out_type=jax.ShapeDtypeStruct((_ROWS, _COLS), jnp.float32),
        mesh=mesh,
        scratch_types=[
            pltpu.VMEM((2, _CHUNK, _COLS), jnp.float32),
            pltpu.SemaphoreType.DMA((2,)),
            pltpu.SemaphoreType.DMA((2,)),
        ],
    )
    def sc_copy(x_hbm, out_hbm, bufs, in_sems, out_sems):
        wid = lax.axis_index("s") * _NC + lax.axis_index("c")
        base = wid * _ROWS_PER

        def in_copy(i):
            return pltpu.make_async_copy(
                x_hbm.at[pl.ds(base + i * _CHUNK, _CHUNK), :],
                bufs.at[i % 2],
                in_sems.at[i % 2],
            )

        def out_copy(i):
            return pltpu.make_async_copy(
                bufs.at[i % 2],
                out_hbm.at[pl.ds(base + i * _CHUNK, _CHUNK), :],
                out_sems.at[i % 2],
            )

        in_copy(0).start()
        for i in range(_NCHUNK):
            in_copy(i).wait()
            out_copy(i).start()
            if i >= 1:
                out_copy(i - 1).wait()
            if i + 1 < _NCHUNK:
                in_copy(i + 1).start()
        out_copy(_NCHUNK - 1).wait()

    return sc_copy(x_j)
